# Initial kernel scaffold; baseline (speedup 1.0000x reference)
import jax
import jax.numpy as jnp
from jax.experimental import pallas as pl


def kernel(x, pos_table):
    raise NotImplementedError("write your pallas kernel here")


# speedup vs baseline: 1.4925x; 1.4925x over previous
import jax
import jax.numpy as jnp
from jax.experimental import pallas as pl

BLOCK_S = 512


def _add_body(x_ref, pos_ref, o_ref):
    o_ref[...] = x_ref[...] + pos_ref[...]


def kernel(x, pos_table):
    batch, seq_len, d_model = x.shape
    grid = (seq_len // BLOCK_S, batch)
    return pl.pallas_call(
        _add_body,
        grid=grid,
        in_specs=[
            pl.BlockSpec((None, BLOCK_S, d_model), lambda s, b: (b, s, 0)),
            pl.BlockSpec((BLOCK_S, d_model), lambda s, b: (s, 0)),
        ],
        out_specs=pl.BlockSpec((None, BLOCK_S, d_model), lambda s, b: (b, s, 0)),
        out_shape=jax.ShapeDtypeStruct(x.shape, x.dtype),
    )(x, pos_table)


# speedup vs baseline: 1.6626x; 1.1139x over previous
import jax
import jax.numpy as jnp
from jax.experimental import pallas as pl

BLOCK_S = 1024


def _add_body(x_ref, pos_ref, o_ref):
    o_ref[...] = x_ref[...] + pos_ref[...]


def kernel(x, pos_table):
    batch, seq_len, d_model = x.shape
    grid = (seq_len // BLOCK_S, batch)
    return pl.pallas_call(
        _add_body,
        grid=grid,
        in_specs=[
            pl.BlockSpec((None, BLOCK_S, d_model), lambda s, b: (b, s, 0)),
            pl.BlockSpec((BLOCK_S, d_model), lambda s, b: (s, 0)),
        ],
        out_specs=pl.BlockSpec((None, BLOCK_S, d_model), lambda s, b: (b, s, 0)),
        out_shape=jax.ShapeDtypeStruct(x.shape, x.dtype),
    )(x, pos_table)


# speedup vs baseline: 1.7360x; 1.0442x over previous
import jax
import jax.numpy as jnp
from jax.experimental import pallas as pl

BLOCK_S = 2048


def _add_body(x_ref, pos_ref, o_ref):
    o_ref[...] = x_ref[...] + pos_ref[...]


def kernel(x, pos_table):
    batch, seq_len, d_model = x.shape
    grid = (seq_len // BLOCK_S, batch)
    return pl.pallas_call(
        _add_body,
        grid=grid,
        in_specs=[
            pl.BlockSpec((None, BLOCK_S, d_model), lambda s, b: (b, s, 0)),
            pl.BlockSpec((BLOCK_S, d_model), lambda s, b: (s, 0)),
        ],
        out_specs=pl.BlockSpec((None, BLOCK_S, d_model), lambda s, b: (b, s, 0)),
        out_shape=jax.ShapeDtypeStruct(x.shape, x.dtype),
    )(x, pos_table)
